# baseline (speedup 1.0000x reference)
import functools

import jax
import jax.numpy as jnp
from jax import lax
from jax.experimental import pallas as pl
from jax.experimental.pallas import tpu as pltpu
from jax.experimental.pallas import tpu_sc as plsc

Q = 1024
N = 16384
D = 1024
K = 4

QB = 256
NB = 2048

_INF = float("inf")
_BIGI = 2**30

_NC = 2
_NS = 16
_NW = _NC * _NS
_QPW = Q // _NW
_GRP = 8


_NJ = N // NB


def _topk_body(nq_ref, nm_ref, q_ref, m_ref, out_ref, cv_ref, ci_ref):
    j = pl.program_id(1)

    dot = lax.dot_general(
        m_ref[...], q_ref[...],
        (((1,), (1,)), ((), ())),
        preferred_element_type=jnp.float32,
    )

    nq = nq_ref[...]
    nm = nm_ref[...]
    nq2 = nq * nq
    nm2 = nm * nm
    cdist_sq = (nq2 + nm2) - 2.0 * dot
    dotprod = (nq2 - cdist_sq) + nm2
    dists = 1.0 - dotprod / ((2.0 * nm) * nq)

    rows = lax.broadcasted_iota(jnp.int32, (NB, QB), 0) + j * NB

    cand_v, cand_i = [], []
    d = dists
    for t in range(K):
        vmin = jnp.min(d, axis=0, keepdims=True)
        imin = jnp.min(jnp.where(d == vmin, rows, _BIGI), axis=0,
                       keepdims=True)
        cand_v.append(vmin)
        cand_i.append(imin)
        if t + 1 < K:
            d = jnp.where(rows == imin, _INF, d)

    for jj in range(_NJ):
        @pl.when(j == jj)
        def _stage(jj=jj):
            for t in range(K):
                s = jj * K + t
                cv_ref[s:s + 1, :] = cand_v[t]
                ci_ref[s:s + 1, :] = cand_i[t]

    @pl.when(j == _NJ - 1)
    def _final():
        vv = cv_ref[...]
        idx = ci_ref[...]
        pos = lax.broadcasted_iota(jnp.int32, (_NJ * K, QB), 0)
        for t in range(K):
            vmin = jnp.min(vv, axis=0, keepdims=True)
            p = jnp.min(jnp.where(vv == vmin, pos, _BIGI), axis=0,
                        keepdims=True)
            hit = pos == p
            sel = jnp.min(jnp.where(hit, idx, _BIGI), axis=0, keepdims=True)
            out_ref[t:t + 1, :] = sel
            if t + 1 < K:
                vv = jnp.where(hit, _INF, vv)


_topk_call = pl.pallas_call(
    _topk_body,
    grid=(Q // QB, N // NB),
    in_specs=[
        pl.BlockSpec((1, QB), lambda i, j: (0, i)),
        pl.BlockSpec((NB, 1), lambda i, j: (j, 0)),
        pl.BlockSpec((QB, D), lambda i, j: (i, 0)),
        pl.BlockSpec((NB, D), lambda i, j: (j, 0)),
    ],
    out_specs=pl.BlockSpec((K, QB), lambda i, j: (0, i)),
    out_shape=jax.ShapeDtypeStruct((K, Q), jnp.int32),
    scratch_shapes=[
        pltpu.VMEM((_NJ * K, QB), jnp.float32),
        pltpu.VMEM((_NJ * K, QB), jnp.int32),
    ],
    compiler_params=pltpu.CompilerParams(
        dimension_semantics=("parallel", "arbitrary"),
    ),
)


def _gather_mean_body(ms_hbm, idx_hbm, out_hbm, idx_v, rows_v, out_v, sem):
    wid = lax.axis_index("s") * _NC + lax.axis_index("c")
    qbase = wid * _QPW
    for g in range(_QPW // _GRP):
        q0 = qbase + g * _GRP
        pltpu.sync_copy(idx_hbm.at[pl.ds(q0 * K, _GRP * K)], idx_v)
        pltpu.async_copy(ms_hbm.at[idx_v], rows_v, sem).wait()

        def _chunk(c, _):
            sl = pl.ds(c * 16, 16)
            for q in range(_GRP):
                s = (rows_v[K * q, sl] + rows_v[K * q + 1, sl]
                     + rows_v[K * q + 2, sl] + rows_v[K * q + 3, sl])
                out_v[q, sl] = s * 0.25
            return _

        lax.fori_loop(0, D // 16, _chunk, None)
        pltpu.sync_copy(out_v, out_hbm.at[pl.ds(q0, _GRP)])


@functools.cache
def _gather_mean_call():
    return pl.kernel(
        _gather_mean_body,
        mesh=plsc.VectorSubcoreMesh(core_axis_name="c", subcore_axis_name="s",
                                    num_cores=_NC, num_subcores=_NS),
        out_type=jax.ShapeDtypeStruct((Q, D), jnp.float32),
        scratch_types=[
            pltpu.VMEM((_GRP * K,), jnp.int32),
            pltpu.VMEM((_GRP * K, D), jnp.float32),
            pltpu.VMEM((_GRP, D), jnp.float32),
            pltpu.SemaphoreType.DMA,
        ],
    )


def kernel(query_seq, matching_set, topk):
    del topk
    nq = jnp.linalg.norm(query_seq, ord=2, axis=-1)
    nm = jnp.linalg.norm(matching_set, ord=2, axis=-1)
    idx = _topk_call(nq.reshape(1, Q), nm.reshape(N, 1),
                     query_seq, matching_set)
    return _gather_mean_call()(matching_set, idx.T.reshape(Q * K))


# speedup vs baseline: 1.0843x; 1.0843x over previous
import functools

import jax
import jax.numpy as jnp
from jax import lax
from jax.experimental import pallas as pl
from jax.experimental.pallas import tpu as pltpu
from jax.experimental.pallas import tpu_sc as plsc

Q = 1024
N = 16384
D = 1024
K = 4

QB = 1024
NB = 2048

_INF = float("inf")
_BIGI = 2**30

_NC = 2
_NS = 16
_NW = _NC * _NS
_QPW = Q // _NW
_GRP = 8


_NJ = N // NB


def _topk_body(nq_ref, nm_ref, q_ref, m_ref, out_ref, cv_ref, ci_ref):
    j = pl.program_id(1)

    dot = lax.dot_general(
        m_ref[...], q_ref[...],
        (((1,), (1,)), ((), ())),
        preferred_element_type=jnp.float32,
    )

    nq = nq_ref[...]
    nm = nm_ref[...]
    nq2 = nq * nq
    nm2 = nm * nm
    cdist_sq = (nq2 + nm2) - 2.0 * dot
    dotprod = (nq2 - cdist_sq) + nm2
    dists = 1.0 - dotprod / ((2.0 * nm) * nq)

    rows = lax.broadcasted_iota(jnp.int32, (NB, QB), 0)

    cand_v, cand_i = [], []
    d = dists
    for t in range(K):
        vmin = jnp.min(d, axis=0, keepdims=True)
        imin = jnp.min(jnp.where(d == vmin, rows, _BIGI), axis=0,
                       keepdims=True)
        cand_v.append(vmin)
        cand_i.append(imin)
        if t + 1 < K:
            d = jnp.where(rows == imin, _INF, d)

    for jj in range(_NJ):
        @pl.when(j == jj)
        def _stage(jj=jj):
            for t in range(K):
                s = jj * K + t
                cv_ref[s:s + 1, :] = cand_v[t]
                ci_ref[s:s + 1, :] = cand_i[t] + jj * NB

    @pl.when(j == _NJ - 1)
    def _final():
        vv = cv_ref[...]
        idx = ci_ref[...]
        pos = lax.broadcasted_iota(jnp.int32, (_NJ * K, QB), 0)
        for t in range(K):
            vmin = jnp.min(vv, axis=0, keepdims=True)
            p = jnp.min(jnp.where(vv == vmin, pos, _BIGI), axis=0,
                        keepdims=True)
            hit = pos == p
            sel = jnp.min(jnp.where(hit, idx, _BIGI), axis=0, keepdims=True)
            out_ref[t:t + 1, :] = sel
            if t + 1 < K:
                vv = jnp.where(hit, _INF, vv)


_topk_call = pl.pallas_call(
    _topk_body,
    grid=(Q // QB, N // NB),
    in_specs=[
        pl.BlockSpec((1, QB), lambda i, j: (0, i)),
        pl.BlockSpec((NB, 1), lambda i, j: (j, 0)),
        pl.BlockSpec((QB, D), lambda i, j: (i, 0)),
        pl.BlockSpec((NB, D), lambda i, j: (j, 0)),
    ],
    out_specs=pl.BlockSpec((K, QB), lambda i, j: (0, i)),
    out_shape=jax.ShapeDtypeStruct((K, Q), jnp.int32),
    scratch_shapes=[
        pltpu.VMEM((_NJ * K, QB), jnp.float32),
        pltpu.VMEM((_NJ * K, QB), jnp.int32),
    ],
    compiler_params=pltpu.CompilerParams(
        dimension_semantics=("parallel", "arbitrary"),
    ),
)


def _gather_mean_body(ms_hbm, idx_hbm, out_hbm, idx_v, rows_v, out_v, sem):
    wid = lax.axis_index("s") * _NC + lax.axis_index("c")
    qbase = wid * _QPW
    for g in range(_QPW // _GRP):
        q0 = qbase + g * _GRP
        pltpu.sync_copy(idx_hbm.at[pl.ds(q0 * K, _GRP * K)], idx_v)
        pltpu.async_copy(ms_hbm.at[idx_v], rows_v, sem).wait()

        def _chunk(c, _):
            sl = pl.ds(c * 16, 16)
            for q in range(_GRP):
                s = (rows_v[K * q, sl] + rows_v[K * q + 1, sl]
                     + rows_v[K * q + 2, sl] + rows_v[K * q + 3, sl])
                out_v[q, sl] = s * 0.25
            return _

        lax.fori_loop(0, D // 16, _chunk, None)
        pltpu.sync_copy(out_v, out_hbm.at[pl.ds(q0, _GRP)])


@functools.cache
def _gather_mean_call():
    return pl.kernel(
        _gather_mean_body,
        mesh=plsc.VectorSubcoreMesh(core_axis_name="c", subcore_axis_name="s",
                                    num_cores=_NC, num_subcores=_NS),
        out_type=jax.ShapeDtypeStruct((Q, D), jnp.float32),
        scratch_types=[
            pltpu.VMEM((_GRP * K,), jnp.int32),
            pltpu.VMEM((_GRP * K, D), jnp.float32),
            pltpu.VMEM((_GRP, D), jnp.float32),
            pltpu.SemaphoreType.DMA,
        ],
    )


def kernel(query_seq, matching_set, topk):
    del topk
    nq = jnp.linalg.norm(query_seq, ord=2, axis=-1)
    nm = jnp.linalg.norm(matching_set, ord=2, axis=-1)
    idx = _topk_call(nq.reshape(1, Q), nm.reshape(N, 1),
                     query_seq, matching_set)
    return _gather_mean_call()(matching_set, idx.T.reshape(Q * K))


# speedup vs baseline: 1.2804x; 1.1809x over previous
import functools

import jax
import jax.numpy as jnp
from jax import lax
from jax.experimental import pallas as pl
from jax.experimental.pallas import tpu as pltpu
from jax.experimental.pallas import tpu_sc as plsc

Q = 1024
N = 16384
D = 1024
K = 4

QB = 1024
NB = 2048

_INF = float("inf")
_BIGI = 2**30

_NC = 2
_NS = 16
_NW = _NC * _NS
_QPW = Q // _NW
_GRP = 8


_NJ = N // NB


def _topk_body(nq_ref, q_ref, m_ref, out_ref, cv_ref, ci_ref):
    j = pl.program_id(1)

    m = m_ref[...]
    dot = lax.dot_general(
        m, q_ref[...],
        (((1,), (1,)), ((), ())),
        preferred_element_type=jnp.float32,
    )

    nq = nq_ref[...]
    nm = jnp.sqrt(jnp.sum(m * m, axis=1, keepdims=True))
    nq2 = nq * nq
    nm2 = nm * nm
    cdist_sq = (nq2 + nm2) - 2.0 * dot
    dotprod = (nq2 - cdist_sq) + nm2
    dists = 1.0 - dotprod / ((2.0 * nm) * nq)

    rows = lax.broadcasted_iota(jnp.int32, (NB, QB), 0)

    cand_v, cand_i = [], []
    d = dists
    for t in range(K):
        vmin = jnp.min(d, axis=0, keepdims=True)
        imin = jnp.min(jnp.where(d == vmin, rows, _BIGI), axis=0,
                       keepdims=True)
        cand_v.append(vmin)
        cand_i.append(imin)
        if t + 1 < K:
            d = jnp.where(rows == imin, _INF, d)

    for jj in range(_NJ):
        @pl.when(j == jj)
        def _stage(jj=jj):
            for t in range(K):
                s = jj * K + t
                cv_ref[s:s + 1, :] = cand_v[t]
                ci_ref[s:s + 1, :] = cand_i[t] + jj * NB

    @pl.when(j == _NJ - 1)
    def _final():
        vv = cv_ref[...]
        idx = ci_ref[...]
        pos = lax.broadcasted_iota(jnp.int32, (_NJ * K, QB), 0)
        for t in range(K):
            vmin = jnp.min(vv, axis=0, keepdims=True)
            p = jnp.min(jnp.where(vv == vmin, pos, _BIGI), axis=0,
                        keepdims=True)
            hit = pos == p
            sel = jnp.min(jnp.where(hit, idx, _BIGI), axis=0, keepdims=True)
            out_ref[t:t + 1, :] = sel
            if t + 1 < K:
                vv = jnp.where(hit, _INF, vv)


_topk_call = pl.pallas_call(
    _topk_body,
    grid=(Q // QB, N // NB),
    in_specs=[
        pl.BlockSpec((1, QB), lambda i, j: (0, i)),
        pl.BlockSpec((QB, D), lambda i, j: (i, 0)),
        pl.BlockSpec((NB, D), lambda i, j: (j, 0)),
    ],
    out_specs=pl.BlockSpec((K, QB), lambda i, j: (0, i)),
    out_shape=jax.ShapeDtypeStruct((K, Q), jnp.int32),
    scratch_shapes=[
        pltpu.VMEM((_NJ * K, QB), jnp.float32),
        pltpu.VMEM((_NJ * K, QB), jnp.int32),
    ],
    compiler_params=pltpu.CompilerParams(
        dimension_semantics=("parallel", "arbitrary"),
    ),
)


_NGRP = _QPW // _GRP


def _gather_mean_body(ms_hbm, idx_hbm, out_hbm, idx_v, rows_v, out_v,
                      gsem0, gsem1, osem0, osem1):
    wid = lax.axis_index("s") * _NC + lax.axis_index("c")
    qbase = wid * _QPW
    pltpu.sync_copy(idx_hbm.at[pl.ds(qbase * K, _QPW * K)], idx_v)
    gsems = (gsem0, gsem1)
    osems = (osem0, osem1)

    def _gather(g):
        sl = g % 2
        return pltpu.async_copy(
            ms_hbm.at[idx_v.at[pl.ds(g * _GRP * K, _GRP * K)]],
            rows_v.at[sl], gsems[sl])

    gh = {0: _gather(0)}
    oh = {}
    for g in range(_NGRP):
        sl = g % 2
        if g + 1 < _NGRP:
            gh[g + 1] = _gather(g + 1)
        gh[g].wait()
        if g - 2 >= 0:
            oh[g - 2].wait()

        def _chunk(c, _, sl=sl):
            cs = pl.ds(c * 16, 16)
            for q in range(_GRP):
                s = (rows_v[sl, K * q, cs] + rows_v[sl, K * q + 1, cs]
                     + rows_v[sl, K * q + 2, cs] + rows_v[sl, K * q + 3, cs])
                out_v[sl, q, cs] = s * 0.25
            return _

        lax.fori_loop(0, D // 16, _chunk, None)
        oh[g] = pltpu.async_copy(
            out_v.at[sl], out_hbm.at[pl.ds(qbase + g * _GRP, _GRP)],
            osems[sl])
    oh[_NGRP - 2].wait()
    oh[_NGRP - 1].wait()


@functools.cache
def _gather_mean_call():
    return pl.kernel(
        _gather_mean_body,
        mesh=plsc.VectorSubcoreMesh(core_axis_name="c", subcore_axis_name="s",
                                    num_cores=_NC, num_subcores=_NS),
        out_type=jax.ShapeDtypeStruct((Q, D), jnp.float32),
        scratch_types=[
            pltpu.VMEM((_QPW * K,), jnp.int32),
            pltpu.VMEM((2, _GRP * K, D), jnp.float32),
            pltpu.VMEM((2, _GRP, D), jnp.float32),
            pltpu.SemaphoreType.DMA,
            pltpu.SemaphoreType.DMA,
            pltpu.SemaphoreType.DMA,
            pltpu.SemaphoreType.DMA,
        ],
    )


def kernel(query_seq, matching_set, topk):
    del topk
    nq = jnp.linalg.norm(query_seq, ord=2, axis=-1)
    idx = _topk_call(nq.reshape(1, Q), query_seq, matching_set)
    return _gather_mean_call()(matching_set, idx.T.reshape(Q * K))
